# Initial kernel scaffold; baseline (speedup 1.0000x reference)
import jax
import jax.numpy as jnp
from jax.experimental import pallas as pl


def kernel(x, edge_index, W1, b1, gamma, beta, W2, b2):
    raise NotImplementedError("write your pallas kernel here")


# speedup vs baseline: 8.3544x; 8.3544x over previous
import functools

import jax
import jax.numpy as jnp
from jax import lax
from jax.experimental import pallas as pl
from jax.experimental.pallas import tpu as pltpu
from jax.experimental.pallas import tpu_sc as plsc

N = 10000
E = 320000
IN = 128
H = 64
C = 64
K = 10
ALPHA = 0.1
EPS = 1e-5

NC = 2
NS = 16
CHUNK = 128
ROWS_PER_TILE = 80
EPAD = NC * NS * ROWS_PER_TILE * CHUNK
NPAD = 10240
NROWS_TILE = NPAD // NS

_mesh = plsc.VectorSubcoreMesh(core_axis_name="c", subcore_axis_name="s")
_sc_params = pltpu.CompilerParams(use_tc_tiling_on_sc=False)


@functools.partial(
    pl.kernel,
    out_type=jax.ShapeDtypeStruct((NC, NPAD), jnp.float32),
    mesh=_mesh,
    compiler_params=_sc_params,
    scratch_types=[
        pltpu.VMEM((ROWS_PER_TILE, CHUNK), jnp.int32),
        pltpu.VMEM((CHUNK,), jnp.float32),
        pltpu.VMEM((NROWS_TILE,), jnp.float32),
        pltpu.VMEM_SHARED((NPAD,), jnp.float32),
    ],
)
def _sc_degree(dst_hbm, ones_hbm, zeros_hbm, out_hbm, dst_v, ones_v, buf_v, dacc):
    cid = lax.axis_index("c")
    sid = lax.axis_index("s")
    wid = cid * NS + sid
    pltpu.sync_copy(dst_hbm.at[pl.ds(wid * ROWS_PER_TILE, ROWS_PER_TILE)], dst_v)
    pltpu.sync_copy(ones_hbm, ones_v)
    pltpu.sync_copy(zeros_hbm, buf_v)
    pltpu.sync_copy(buf_v, dacc.at[pl.ds(sid * NROWS_TILE, NROWS_TILE)])
    plsc.subcore_barrier()

    def body(j, carry):
        pltpu.sync_copy(ones_v, dacc.at[dst_v.at[j]], add=True)
        return carry

    lax.fori_loop(0, ROWS_PER_TILE, body, 0)
    plsc.subcore_barrier()
    pltpu.sync_copy(dacc.at[pl.ds(sid * NROWS_TILE, NROWS_TILE)], buf_v)
    pltpu.sync_copy(buf_v, out_hbm.at[cid, pl.ds(sid * NROWS_TILE, NROWS_TILE)])


@functools.partial(
    pl.kernel,
    out_type=jax.ShapeDtypeStruct((NC, NPAD, H), jnp.float32),
    mesh=_mesh,
    compiler_params=_sc_params,
    scratch_types=[
        pltpu.VMEM((ROWS_PER_TILE, CHUNK), jnp.int32),
        pltpu.VMEM((ROWS_PER_TILE, CHUNK), jnp.int32),
        pltpu.VMEM((CHUNK, H), jnp.float32),
        pltpu.VMEM((CHUNK, H), jnp.float32),
        pltpu.VMEM_SHARED((NPAD, H), jnp.float32),
        pltpu.SemaphoreType.DMA,
    ],
)
def _sc_propagate(u_hbm, src_hbm, dst_hbm, zrows_hbm, out_hbm,
                  src_v, dst_v, rows_v, z_v, acc, sem):
    cid = lax.axis_index("c")
    sid = lax.axis_index("s")
    wid = cid * NS + sid
    pltpu.sync_copy(src_hbm.at[pl.ds(wid * ROWS_PER_TILE, ROWS_PER_TILE)], src_v)
    pltpu.sync_copy(dst_hbm.at[pl.ds(wid * ROWS_PER_TILE, ROWS_PER_TILE)], dst_v)
    pltpu.sync_copy(zrows_hbm, z_v)
    for b in range(NROWS_TILE // CHUNK):
        pltpu.sync_copy(z_v, acc.at[pl.ds(sid * NROWS_TILE + b * CHUNK, CHUNK)])
    plsc.subcore_barrier()

    def body(j, carry):
        pltpu.async_copy(u_hbm.at[src_v.at[j]], rows_v, sem).wait()
        pltpu.sync_copy(rows_v, acc.at[dst_v.at[j]], add=True)
        return carry

    lax.fori_loop(0, ROWS_PER_TILE, body, 0)
    plsc.subcore_barrier()
    for b in range(NROWS_TILE // CHUNK):
        r0 = sid * NROWS_TILE + b * CHUNK
        pltpu.sync_copy(acc.at[pl.ds(r0, CHUNK)], rows_v)
        pltpu.sync_copy(rows_v, out_hbm.at[cid, pl.ds(r0, CHUNK)])


def _tc_mm1(x, W1):
    def body(x_ref, w_ref, o_ref):
        o_ref[...] = jnp.dot(x_ref[...], w_ref[...],
                             preferred_element_type=jnp.float32)
    return pl.pallas_call(
        body, out_shape=jax.ShapeDtypeStruct((N, H), jnp.float32))(x, W1)


def _tc_prep(dega, degb, hp):
    def body(da, db, hp_ref, v0, dinv, dinv2, sqd):
        deg = da[...] + db[...] + 1.0
        di = lax.rsqrt(deg)
        dinv[...] = di
        dinv2[...] = 1.0 / deg
        sqd[...] = jnp.sqrt(deg)
        v0[...] = hp_ref[...] * di[:, None]
    return pl.pallas_call(
        body,
        out_shape=(
            jax.ShapeDtypeStruct((N, H), jnp.float32),
            jax.ShapeDtypeStruct((N,), jnp.float32),
            jax.ShapeDtypeStruct((N,), jnp.float32),
            jax.ShapeDtypeStruct((N,), jnp.float32),
        ))(dega, degb, hp)


def _tc_bn(acc_a, acc_b, v0, dinv, b1, gamma, beta):
    def body(aa, ab, v0r, di, b1r, gr, br, u0r):
        h1 = di[...][:, None] * (aa[...] + ab[...] + v0r[...]) + b1r[...][None, :]
        mean = jnp.mean(h1, axis=0, keepdims=True)
        var = jnp.mean((h1 - mean) ** 2, axis=0, keepdims=True)
        h0 = (h1 - mean) * lax.rsqrt(var + EPS) * gr[...][None, :] + br[...][None, :]
        h0 = jnp.maximum(h0, 0.0)
        u0r[...] = di[...][:, None] * h0
    return pl.pallas_call(
        body, out_shape=jax.ShapeDtypeStruct((N, H), jnp.float32))(
            acc_a, acc_b, v0, dinv, b1, gamma, beta)


def _tc_combine(acc_a, acc_b, u, u0, dinv2):
    def body(aa, ab, ur, u0r, d2, out):
        out[...] = ((1.0 - ALPHA) * d2[...][:, None]
                    * (aa[...] + ab[...] + ur[...]) + ALPHA * u0r[...])
    return pl.pallas_call(
        body, out_shape=jax.ShapeDtypeStruct((N, H), jnp.float32))(
            acc_a, acc_b, u, u0, dinv2)


def _tc_final(u, sqd, W2, b2):
    def body(ur, sr, w_ref, b_ref, out):
        z = sr[...][:, None] * ur[...]
        o = jnp.dot(z, w_ref[...], preferred_element_type=jnp.float32)
        o = o + b_ref[...][None, :]
        o = o - jnp.max(o, axis=1, keepdims=True)
        out[...] = o - jnp.log(jnp.sum(jnp.exp(o), axis=1, keepdims=True))
    return pl.pallas_call(
        body, out_shape=jax.ShapeDtypeStruct((N, C), jnp.float32))(
            u, sqd, W2, b2)


def kernel(x, edge_index, W1, b1, gamma, beta, W2, b2):
    src = edge_index[0]
    dst = edge_index[1]
    pad = EPAD - E
    src2d = jnp.concatenate(
        [src, jnp.zeros((pad,), jnp.int32)]).reshape(-1, CHUNK)
    dst2d = jnp.concatenate(
        [dst, jnp.full((pad,), N, jnp.int32)]).reshape(-1, CHUNK)
    zrows = jnp.zeros((CHUNK, H), jnp.float32)
    zeros_t = jnp.zeros((NROWS_TILE,), jnp.float32)
    ones_c = jnp.ones((CHUNK,), jnp.float32)

    hp = _tc_mm1(x, W1)
    degs = _sc_degree(dst2d, ones_c, zeros_t)
    v0, dinv, dinv2, sqd = _tc_prep(degs[0, :N], degs[1, :N], hp)
    acc = _sc_propagate(v0, src2d, dst2d, zrows)
    u = _tc_bn(acc[0, :N], acc[1, :N], v0, dinv, b1, gamma, beta)
    u0 = u
    for _ in range(K):
        acc = _sc_propagate(u, src2d, dst2d, zrows)
        u = _tc_combine(acc[0, :N], acc[1, :N], u, u0, dinv2)
    return _tc_final(u, sqd, W2, b2)


# speedup vs baseline: 10.0072x; 1.1978x over previous
import functools

import jax
import jax.numpy as jnp
from jax import lax
from jax.experimental import pallas as pl
from jax.experimental.pallas import tpu as pltpu
from jax.experimental.pallas import tpu_sc as plsc

N = 10000
E = 320000
IN = 128
H = 64
C = 64
K = 10
ALPHA = 0.1
EPS = 1e-5

NC = 2
NS = 16
ZB = 32
GW = 512
GROUPS = 20
EPT = GW * GROUPS
EPAD = NC * NS * EPT
NPAD = 10240
NROWS_TILE = NPAD // NS

_mesh = plsc.VectorSubcoreMesh(core_axis_name="c", subcore_axis_name="s")
_sc_params = pltpu.CompilerParams(use_tc_tiling_on_sc=False)


@functools.partial(
    pl.kernel,
    out_type=jax.ShapeDtypeStruct((NC, NPAD), jnp.float32),
    mesh=_mesh,
    compiler_params=_sc_params,
    scratch_types=[
        pltpu.VMEM((GROUPS, GW), jnp.int32),
        pltpu.VMEM((GW,), jnp.float32),
        pltpu.VMEM((NROWS_TILE,), jnp.float32),
        pltpu.VMEM_SHARED((NPAD,), jnp.float32),
        pltpu.SemaphoreType.DMA,
    ],
)
def _sc_degree(dst_hbm, ones_hbm, zeros_hbm, out_hbm,
               dst_v, ones_v, buf_v, dacc, sem):
    cid = lax.axis_index("c")
    sid = lax.axis_index("s")
    wid = cid * NS + sid
    pltpu.sync_copy(dst_hbm.at[pl.ds(wid * GROUPS, GROUPS)], dst_v)
    pltpu.sync_copy(ones_hbm, ones_v)
    pltpu.sync_copy(zeros_hbm, buf_v)
    pltpu.sync_copy(buf_v, dacc.at[pl.ds(sid * NROWS_TILE, NROWS_TILE)])
    plsc.subcore_barrier()
    descs = []
    for g in range(GROUPS):
        descs.append(pltpu.async_copy(
            ones_v, dacc.at[dst_v.at[g]], sem, add=True))
    for d in descs:
        d.wait()
    plsc.subcore_barrier()
    pltpu.sync_copy(dacc.at[pl.ds(sid * NROWS_TILE, NROWS_TILE)], buf_v)
    pltpu.sync_copy(buf_v, out_hbm.at[cid, pl.ds(sid * NROWS_TILE, NROWS_TILE)])


@functools.partial(
    pl.kernel,
    out_type=jax.ShapeDtypeStruct((NC, NPAD, H), jnp.float32),
    mesh=_mesh,
    compiler_params=_sc_params,
    scratch_types=[
        pltpu.VMEM((GROUPS, GW), jnp.int32),
        pltpu.VMEM((GROUPS, GW), jnp.int32),
        pltpu.VMEM((GW, H), jnp.float32),
        pltpu.VMEM((GW, H), jnp.float32),
        pltpu.VMEM((ZB, H), jnp.float32),
        pltpu.VMEM_SHARED((NPAD, H), jnp.float32),
        pltpu.SemaphoreType.DMA,
        pltpu.SemaphoreType.DMA,
        pltpu.SemaphoreType.DMA,
    ],
)
def _sc_propagate(u_hbm, src_hbm, dst_hbm, zrows_hbm, out_hbm,
                  src_v, dst_v, buf0, buf1, z_v, acc, sem_g, sem_s0, sem_s1):
    cid = lax.axis_index("c")
    sid = lax.axis_index("s")
    wid = cid * NS + sid
    pltpu.sync_copy(src_hbm.at[pl.ds(wid * GROUPS, GROUPS)], src_v)
    pltpu.sync_copy(dst_hbm.at[pl.ds(wid * GROUPS, GROUPS)], dst_v)
    pltpu.sync_copy(zrows_hbm, z_v)
    for b in range(NROWS_TILE // ZB):
        pltpu.sync_copy(z_v, acc.at[pl.ds(sid * NROWS_TILE + b * ZB, ZB)])
    plsc.subcore_barrier()

    bufs = (buf0, buf1)
    ssems = (sem_s0, sem_s1)

    def gather(g):
        return pltpu.async_copy(
            u_hbm.at[src_v.at[g]], bufs[g % 2], sem_g)

    def scatter(g):
        return pltpu.async_copy(
            bufs[g % 2], acc.at[dst_v.at[g]], ssems[g % 2], add=True)

    gd = gather(0)
    sd = [None] * GROUPS
    for g in range(GROUPS):
        gd.wait()
        sd[g] = scatter(g)
        if g + 1 < GROUPS:
            if g >= 1:
                sd[g - 1].wait()
            gd = gather(g + 1)
    if GROUPS >= 2:
        sd[GROUPS - 2].wait()
    sd[GROUPS - 1].wait()

    plsc.subcore_barrier()
    r0 = sid * NROWS_TILE
    pltpu.sync_copy(acc.at[pl.ds(r0, GW)], buf0)
    pltpu.sync_copy(buf0, out_hbm.at[cid, pl.ds(r0, GW)])
    rest = NROWS_TILE - GW
    pltpu.sync_copy(acc.at[pl.ds(r0 + GW, rest)], buf1.at[pl.ds(0, rest)])
    pltpu.sync_copy(buf1.at[pl.ds(0, rest)], out_hbm.at[cid, pl.ds(r0 + GW, rest)])


def _tc_mm1(x, W1):
    def body(x_ref, w_ref, o_ref):
        o_ref[...] = jnp.dot(x_ref[...], w_ref[...],
                             preferred_element_type=jnp.float32)
    return pl.pallas_call(
        body, out_shape=jax.ShapeDtypeStruct((N, H), jnp.float32))(x, W1)


def _tc_prep(dega, degb, hp):
    def body(da, db, hp_ref, v0, dinv, dinv2, sqd):
        deg = da[...] + db[...] + 1.0
        di = lax.rsqrt(deg)
        dinv[...] = di
        dinv2[...] = 1.0 / deg
        sqd[...] = jnp.sqrt(deg)
        v0[...] = hp_ref[...] * di[:, None]
    return pl.pallas_call(
        body,
        out_shape=(
            jax.ShapeDtypeStruct((N, H), jnp.float32),
            jax.ShapeDtypeStruct((N,), jnp.float32),
            jax.ShapeDtypeStruct((N,), jnp.float32),
            jax.ShapeDtypeStruct((N,), jnp.float32),
        ))(dega, degb, hp)


def _tc_bn(acc_a, acc_b, v0, dinv, b1, gamma, beta):
    def body(aa, ab, v0r, di, b1r, gr, br, u0r):
        h1 = di[...][:, None] * (aa[...] + ab[...] + v0r[...]) + b1r[...][None, :]
        mean = jnp.mean(h1, axis=0, keepdims=True)
        var = jnp.mean((h1 - mean) ** 2, axis=0, keepdims=True)
        h0 = (h1 - mean) * lax.rsqrt(var + EPS) * gr[...][None, :] + br[...][None, :]
        h0 = jnp.maximum(h0, 0.0)
        u0r[...] = di[...][:, None] * h0
    return pl.pallas_call(
        body, out_shape=jax.ShapeDtypeStruct((N, H), jnp.float32))(
            acc_a, acc_b, v0, dinv, b1, gamma, beta)


def _tc_combine(acc_a, acc_b, u, u0, dinv2):
    def body(aa, ab, ur, u0r, d2, out):
        out[...] = ((1.0 - ALPHA) * d2[...][:, None]
                    * (aa[...] + ab[...] + ur[...]) + ALPHA * u0r[...])
    return pl.pallas_call(
        body, out_shape=jax.ShapeDtypeStruct((N, H), jnp.float32))(
            acc_a, acc_b, u, u0, dinv2)


def _tc_final(u, sqd, W2, b2):
    def body(ur, sr, w_ref, b_ref, out):
        z = sr[...][:, None] * ur[...]
        o = jnp.dot(z, w_ref[...], preferred_element_type=jnp.float32)
        o = o + b_ref[...][None, :]
        o = o - jnp.max(o, axis=1, keepdims=True)
        out[...] = o - jnp.log(jnp.sum(jnp.exp(o), axis=1, keepdims=True))
    return pl.pallas_call(
        body, out_shape=jax.ShapeDtypeStruct((N, C), jnp.float32))(
            u, sqd, W2, b2)


def kernel(x, edge_index, W1, b1, gamma, beta, W2, b2):
    src = edge_index[0]
    dst = edge_index[1]
    pad = EPAD - E
    src2d = jnp.concatenate(
        [src, jnp.zeros((pad,), jnp.int32)]).reshape(-1, GW)
    dst2d = jnp.concatenate(
        [dst, jnp.full((pad,), N, jnp.int32)]).reshape(-1, GW)
    zrows = jnp.zeros((ZB, H), jnp.float32)
    zeros_t = jnp.zeros((NROWS_TILE,), jnp.float32)
    ones_c = jnp.ones((GW,), jnp.float32)

    hp = _tc_mm1(x, W1)
    degs = _sc_degree(dst2d, ones_c, zeros_t)
    v0, dinv, dinv2, sqd = _tc_prep(degs[0, :N], degs[1, :N], hp)
    acc = _sc_propagate(v0, src2d, dst2d, zrows)
    u = _tc_bn(acc[0, :N], acc[1, :N], v0, dinv, b1, gamma, beta)
    u0 = u
    for _ in range(K):
        acc = _sc_propagate(u, src2d, dst2d, zrows)
        u = _tc_combine(acc[0, :N], acc[1, :N], u, u0, dinv2)
    return _tc_final(u, sqd, W2, b2)


# speedup vs baseline: 23.2769x; 2.3260x over previous
import functools

import jax
import jax.numpy as jnp
from jax import lax
from jax.experimental import pallas as pl
from jax.experimental.pallas import tpu as pltpu
from jax.experimental.pallas import tpu_sc as plsc

N = 10000
E = 320000
IN = 128
H = 64
C = 64
K = 10
ALPHA = 0.1
EPS = 1e-5

NC = 2
NS = 16
GW = 320
GROUPS = 32
EPT = GW * GROUPS
EPAD = NC * NS * EPT
NPAD = 10240
NROWS_TILE = NPAD // NS
ZB = 32
UROWS = N // NS
U1 = 320
U2 = UROWS - U1

_mesh = plsc.VectorSubcoreMesh(core_axis_name="c", subcore_axis_name="s")
_sc_params = pltpu.CompilerParams(use_tc_tiling_on_sc=False)


@functools.partial(
    pl.kernel,
    out_type=jax.ShapeDtypeStruct((NC, NPAD), jnp.float32),
    mesh=_mesh,
    compiler_params=_sc_params,
    scratch_types=[
        pltpu.VMEM((GROUPS, 2, GW), jnp.int32),
        pltpu.VMEM((GW,), jnp.float32),
        pltpu.VMEM((NROWS_TILE,), jnp.float32),
        pltpu.VMEM_SHARED((NPAD,), jnp.float32),
        pltpu.SemaphoreType.DMA,
    ],
)
def _sc_degree(idx_hbm, ones_hbm, zeros_hbm, out_hbm,
               idx_v, ones_v, buf_v, dacc, sem):
    cid = lax.axis_index("c")
    sid = lax.axis_index("s")
    wid = cid * NS + sid
    pltpu.sync_copy(idx_hbm.at[pl.ds(wid * GROUPS, GROUPS)], idx_v)
    pltpu.sync_copy(ones_hbm, ones_v)
    pltpu.sync_copy(zeros_hbm, buf_v)
    pltpu.sync_copy(buf_v, dacc.at[pl.ds(sid * NROWS_TILE, NROWS_TILE)])
    plsc.subcore_barrier()
    descs = []
    for g in range(GROUPS):
        descs.append(pltpu.async_copy(
            ones_v, dacc.at[idx_v.at[g, 1]], sem, add=True))
    for d in descs:
        d.wait()
    plsc.subcore_barrier()
    pltpu.sync_copy(dacc.at[pl.ds(sid * NROWS_TILE, NROWS_TILE)], buf_v)
    pltpu.sync_copy(buf_v, out_hbm.at[cid, pl.ds(sid * NROWS_TILE, NROWS_TILE)])


@functools.partial(
    pl.kernel,
    out_type=jax.ShapeDtypeStruct((NC, NPAD, H), jnp.float32),
    mesh=_mesh,
    compiler_params=_sc_params,
    scratch_types=[
        pltpu.VMEM((3, 2, GW), jnp.int32),
        pltpu.VMEM((GW, H), jnp.float32),
        pltpu.VMEM((GW, H), jnp.float32),
        pltpu.VMEM((ZB, H), jnp.float32),
        pltpu.VMEM_SHARED((N, H), jnp.float32),
        pltpu.VMEM_SHARED((NPAD, H), jnp.float32),
        pltpu.SemaphoreType.DMA,
        pltpu.SemaphoreType.DMA,
        pltpu.SemaphoreType.DMA,
        pltpu.SemaphoreType.DMA,
        pltpu.SemaphoreType.DMA,
        pltpu.SemaphoreType.DMA,
    ],
)
def _sc_propagate(u_hbm, idx_hbm, zrows_hbm, out_hbm,
                  idx_v, buf0, buf1, z_v, u_sp, acc,
                  sem_g, ss0, ss1, si0, si1, si2):
    cid = lax.axis_index("c")
    sid = lax.axis_index("s")
    wid = cid * NS + sid
    base = wid * GROUPS
    pltpu.sync_copy(u_hbm.at[pl.ds(sid * UROWS, U1)], buf0)
    pltpu.sync_copy(buf0, u_sp.at[pl.ds(sid * UROWS, U1)])
    pltpu.sync_copy(u_hbm.at[pl.ds(sid * UROWS + U1, U2)], buf1.at[pl.ds(0, U2)])
    pltpu.sync_copy(buf1.at[pl.ds(0, U2)], u_sp.at[pl.ds(sid * UROWS + U1, U2)])
    pltpu.sync_copy(zrows_hbm, z_v)
    for b in range(NROWS_TILE // ZB):
        pltpu.sync_copy(z_v, acc.at[pl.ds(sid * NROWS_TILE + b * ZB, ZB)])
    pltpu.sync_copy(idx_hbm.at[base + 0], idx_v.at[0])
    pltpu.sync_copy(idx_hbm.at[base + 1], idx_v.at[1])
    plsc.subcore_barrier()

    bufs = (buf0, buf1)
    ssems = (ss0, ss1)
    isems = (si0, si1, si2)

    def gather(g):
        return pltpu.async_copy(u_sp.at[idx_v.at[g % 3, 0]], bufs[g % 2], sem_g)

    def scatter(g):
        return pltpu.async_copy(
            bufs[g % 2], acc.at[idx_v.at[g % 3, 1]], ssems[g % 2], add=True)

    def prefetch(g):
        return pltpu.async_copy(
            idx_hbm.at[base + g], idx_v.at[g % 3], isems[g % 3])

    gd = gather(0)
    sd = [None] * GROUPS
    pd = [None] * GROUPS
    for g in range(GROUPS):
        gd.wait()
        sd[g] = scatter(g)
        if g >= 1:
            sd[g - 1].wait()
        if g + 2 < GROUPS:
            pd[g + 2] = prefetch(g + 2)
        if g + 1 < GROUPS:
            if pd[g + 1] is not None:
                pd[g + 1].wait()
            gd = gather(g + 1)
    sd[GROUPS - 1].wait()

    plsc.subcore_barrier()
    r0 = sid * NROWS_TILE
    pltpu.sync_copy(acc.at[pl.ds(r0, GW)], buf0)
    pltpu.sync_copy(buf0, out_hbm.at[cid, pl.ds(r0, GW)])
    pltpu.sync_copy(acc.at[pl.ds(r0 + GW, GW)], buf1)
    pltpu.sync_copy(buf1, out_hbm.at[cid, pl.ds(r0 + GW, GW)])


def _tc_mm1(x, W1):
    def body(x_ref, w_ref, o_ref):
        o_ref[...] = jnp.dot(x_ref[...], w_ref[...],
                             preferred_element_type=jnp.float32)
    return pl.pallas_call(
        body, out_shape=jax.ShapeDtypeStruct((N, H), jnp.float32))(x, W1)


def _tc_prep(dega, degb, hp):
    def body(da, db, hp_ref, v0, dinv, dinv2, sqd):
        deg = da[...] + db[...] + 1.0
        di = lax.rsqrt(deg)
        dinv[...] = di
        dinv2[...] = 1.0 / deg
        sqd[...] = jnp.sqrt(deg)
        v0[...] = hp_ref[...] * di[:, None]
    return pl.pallas_call(
        body,
        out_shape=(
            jax.ShapeDtypeStruct((N, H), jnp.float32),
            jax.ShapeDtypeStruct((N,), jnp.float32),
            jax.ShapeDtypeStruct((N,), jnp.float32),
            jax.ShapeDtypeStruct((N,), jnp.float32),
        ))(dega, degb, hp)


def _tc_bn(acc_a, acc_b, v0, dinv, b1, gamma, beta):
    def body(aa, ab, v0r, di, b1r, gr, br, u0r):
        h1 = di[...][:, None] * (aa[...] + ab[...] + v0r[...]) + b1r[...][None, :]
        mean = jnp.mean(h1, axis=0, keepdims=True)
        var = jnp.mean((h1 - mean) ** 2, axis=0, keepdims=True)
        h0 = (h1 - mean) * lax.rsqrt(var + EPS) * gr[...][None, :] + br[...][None, :]
        h0 = jnp.maximum(h0, 0.0)
        u0r[...] = di[...][:, None] * h0
    return pl.pallas_call(
        body, out_shape=jax.ShapeDtypeStruct((N, H), jnp.float32))(
            acc_a, acc_b, v0, dinv, b1, gamma, beta)


def _tc_combine(acc_a, acc_b, u, u0, dinv2):
    def body(aa, ab, ur, u0r, d2, out):
        out[...] = ((1.0 - ALPHA) * d2[...][:, None]
                    * (aa[...] + ab[...] + ur[...]) + ALPHA * u0r[...])
    return pl.pallas_call(
        body, out_shape=jax.ShapeDtypeStruct((N, H), jnp.float32))(
            acc_a, acc_b, u, u0, dinv2)


def _tc_final(u, sqd, W2, b2):
    def body(ur, sr, w_ref, b_ref, out):
        z = sr[...][:, None] * ur[...]
        o = jnp.dot(z, w_ref[...], preferred_element_type=jnp.float32)
        o = o + b_ref[...][None, :]
        o = o - jnp.max(o, axis=1, keepdims=True)
        out[...] = o - jnp.log(jnp.sum(jnp.exp(o), axis=1, keepdims=True))
    return pl.pallas_call(
        body, out_shape=jax.ShapeDtypeStruct((N, C), jnp.float32))(
            u, sqd, W2, b2)


def kernel(x, edge_index, W1, b1, gamma, beta, W2, b2):
    src = edge_index[0]
    dst = edge_index[1]
    pad = EPAD - E
    src2d = jnp.concatenate(
        [src, jnp.zeros((pad,), jnp.int32)]).reshape(-1, GW)
    dst2d = jnp.concatenate(
        [dst, jnp.full((pad,), N, jnp.int32)]).reshape(-1, GW)
    idx3 = jnp.stack([src2d, dst2d], axis=1)
    zrows = jnp.zeros((ZB, H), jnp.float32)
    zeros_t = jnp.zeros((NROWS_TILE,), jnp.float32)
    ones_c = jnp.ones((GW,), jnp.float32)

    hp = _tc_mm1(x, W1)
    degs = _sc_degree(idx3, ones_c, zeros_t)
    v0, dinv, dinv2, sqd = _tc_prep(degs[0, :N], degs[1, :N], hp)
    acc = _sc_propagate(v0, idx3, zrows)
    u = _tc_bn(acc[0, :N], acc[1, :N], v0, dinv, b1, gamma, beta)
    u0 = u
    for _ in range(K):
        acc = _sc_propagate(u, idx3, zrows)
        u = _tc_combine(acc[0, :N], acc[1, :N], u, u0, dinv2)
    return _tc_final(u, sqd, W2, b2)


# speedup vs baseline: 30.5682x; 1.3132x over previous
import functools

import jax
import jax.numpy as jnp
from jax import lax
from jax.experimental import pallas as pl
from jax.experimental.pallas import tpu as pltpu
from jax.experimental.pallas import tpu_sc as plsc

N = 10000
E = 320000
IN = 128
H = 64
C = 64
K = 10
ALPHA = 0.1
EPS = 1e-5

NC = 2
NS = 16
GW = 320
GROUPS = 32
EPT = GW * GROUPS
EPAD = NC * NS * EPT
NPAD = 10240
NROWS_TILE = NPAD // NS
ZB = 32
UROWS = N // NS
U1 = 320
U2 = UROWS - U1
HH = H // 2
GW2 = 512
GROUPS2 = 40

_mesh = plsc.VectorSubcoreMesh(core_axis_name="c", subcore_axis_name="s")
_sc_params = pltpu.CompilerParams(use_tc_tiling_on_sc=False)


@functools.partial(
    pl.kernel,
    out_type=jax.ShapeDtypeStruct((NC, NPAD), jnp.float32),
    mesh=_mesh,
    compiler_params=_sc_params,
    scratch_types=[
        pltpu.VMEM((GROUPS, 2, GW), jnp.int32),
        pltpu.VMEM((GW,), jnp.float32),
        pltpu.VMEM((NROWS_TILE,), jnp.float32),
        pltpu.VMEM_SHARED((NPAD,), jnp.float32),
        pltpu.SemaphoreType.DMA,
    ],
)
def _sc_degree(idx_hbm, ones_hbm, zeros_hbm, out_hbm,
               idx_v, ones_v, buf_v, dacc, sem):
    cid = lax.axis_index("c")
    sid = lax.axis_index("s")
    wid = cid * NS + sid
    pltpu.sync_copy(idx_hbm.at[pl.ds(wid * GROUPS, GROUPS)], idx_v)
    pltpu.sync_copy(ones_hbm, ones_v)
    pltpu.sync_copy(zeros_hbm, buf_v)
    pltpu.sync_copy(buf_v, dacc.at[pl.ds(sid * NROWS_TILE, NROWS_TILE)])
    plsc.subcore_barrier()
    descs = []
    for g in range(GROUPS):
        descs.append(pltpu.async_copy(
            ones_v, dacc.at[idx_v.at[g, 1]], sem, add=True))
    for d in descs:
        d.wait()
    plsc.subcore_barrier()
    pltpu.sync_copy(dacc.at[pl.ds(sid * NROWS_TILE, NROWS_TILE)], buf_v)
    pltpu.sync_copy(buf_v, out_hbm.at[cid, pl.ds(sid * NROWS_TILE, NROWS_TILE)])


@functools.partial(
    pl.kernel,
    out_type=jax.ShapeDtypeStruct((NC, NPAD, H), jnp.float32),
    mesh=_mesh,
    compiler_params=_sc_params,
    scratch_types=[
        pltpu.VMEM((3, 2, GW), jnp.int32),
        pltpu.VMEM((GW, H), jnp.float32),
        pltpu.VMEM((GW, H), jnp.float32),
        pltpu.VMEM((ZB, H), jnp.float32),
        pltpu.VMEM_SHARED((N, H), jnp.float32),
        pltpu.VMEM_SHARED((NPAD, H), jnp.float32),
        pltpu.SemaphoreType.DMA,
        pltpu.SemaphoreType.DMA,
        pltpu.SemaphoreType.DMA,
        pltpu.SemaphoreType.DMA,
        pltpu.SemaphoreType.DMA,
        pltpu.SemaphoreType.DMA,
    ],
)
def _sc_propagate(u_hbm, idx_hbm, zrows_hbm, out_hbm,
                  idx_v, buf0, buf1, z_v, u_sp, acc,
                  sem_g, ss0, ss1, si0, si1, si2):
    cid = lax.axis_index("c")
    sid = lax.axis_index("s")
    wid = cid * NS + sid
    base = wid * GROUPS
    pltpu.sync_copy(u_hbm.at[pl.ds(sid * UROWS, U1)], buf0)
    pltpu.sync_copy(buf0, u_sp.at[pl.ds(sid * UROWS, U1)])
    pltpu.sync_copy(u_hbm.at[pl.ds(sid * UROWS + U1, U2)], buf1.at[pl.ds(0, U2)])
    pltpu.sync_copy(buf1.at[pl.ds(0, U2)], u_sp.at[pl.ds(sid * UROWS + U1, U2)])
    pltpu.sync_copy(zrows_hbm, z_v)
    for b in range(NROWS_TILE // ZB):
        pltpu.sync_copy(z_v, acc.at[pl.ds(sid * NROWS_TILE + b * ZB, ZB)])
    pltpu.sync_copy(idx_hbm.at[base + 0], idx_v.at[0])
    pltpu.sync_copy(idx_hbm.at[base + 1], idx_v.at[1])
    plsc.subcore_barrier()

    bufs = (buf0, buf1)
    ssems = (ss0, ss1)
    isems = (si0, si1, si2)

    def gather(g):
        return pltpu.async_copy(u_sp.at[idx_v.at[g % 3, 0]], bufs[g % 2], sem_g)

    def scatter(g):
        return pltpu.async_copy(
            bufs[g % 2], acc.at[idx_v.at[g % 3, 1]], ssems[g % 2], add=True)

    def prefetch(g):
        return pltpu.async_copy(
            idx_hbm.at[base + g], idx_v.at[g % 3], isems[g % 3])

    gd = gather(0)
    sd = [None] * GROUPS
    pd = [None] * GROUPS
    for g in range(GROUPS):
        gd.wait()
        sd[g] = scatter(g)
        if g >= 1:
            sd[g - 1].wait()
        if g + 2 < GROUPS:
            pd[g + 2] = prefetch(g + 2)
        if g + 1 < GROUPS:
            if pd[g + 1] is not None:
                pd[g + 1].wait()
            gd = gather(g + 1)
    sd[GROUPS - 1].wait()

    plsc.subcore_barrier()
    r0 = sid * NROWS_TILE
    pltpu.sync_copy(acc.at[pl.ds(r0, GW)], buf0)
    pltpu.sync_copy(buf0, out_hbm.at[cid, pl.ds(r0, GW)])
    pltpu.sync_copy(acc.at[pl.ds(r0 + GW, GW)], buf1)
    pltpu.sync_copy(buf1, out_hbm.at[cid, pl.ds(r0 + GW, GW)])


@functools.partial(
    pl.kernel,
    out_type=jax.ShapeDtypeStruct((NC, N, HH), jnp.float32),
    mesh=_mesh,
    compiler_params=_sc_params,
    scratch_types=[
        pltpu.VMEM((3, 2, GW2), jnp.int32),
        pltpu.VMEM((2 * U1, HH), jnp.float32),
        pltpu.VMEM((2 * U1, HH), jnp.float32),
        pltpu.VMEM((ZB, HH), jnp.float32),
        pltpu.VMEM_SHARED((N, HH), jnp.float32),
        pltpu.VMEM_SHARED((N, HH), jnp.float32),
        pltpu.VMEM_SHARED((N, HH), jnp.float32),
        pltpu.VMEM_SHARED((NPAD, HH), jnp.float32),
        pltpu.SemaphoreType.DMA,
        pltpu.SemaphoreType.DMA,
        pltpu.SemaphoreType.DMA,
        pltpu.SemaphoreType.DMA,
        pltpu.SemaphoreType.DMA,
        pltpu.SemaphoreType.DMA,
    ],
)
def _sc_appnp(u0_hbm, d2_hbm, idx_hbm, zrows_hbm, out_hbm,
              idx_v, buf0, buf1, z_v, u_sp, u0_sp, d2_sp, acc,
              sem_g, ss0, ss1, si0, si1, si2):
    cid = lax.axis_index("c")
    sid = lax.axis_index("s")
    base = sid * GROUPS2
    r0 = sid * UROWS

    def stage(src_hbm, *dsts):
        for (o, l) in ((0, U1), (U1, U2)):
            pltpu.sync_copy(src_hbm.at[pl.ds(r0 + o, l)], buf0.at[pl.ds(0, l)])
            for d in dsts:
                pltpu.sync_copy(buf0.at[pl.ds(0, l)], d.at[pl.ds(r0 + o, l)])

    stage(u0_hbm.at[cid], u_sp, u0_sp)
    stage(d2_hbm, d2_sp)
    pltpu.sync_copy(zrows_hbm, z_v)
    for b in range(NROWS_TILE // ZB):
        pltpu.sync_copy(z_v, acc.at[pl.ds(sid * NROWS_TILE + b * ZB, ZB)])
    plsc.subcore_barrier()

    bufs = (buf0, buf1)
    ssems = (ss0, ss1)
    isems = (si0, si1, si2)

    def gather(g):
        return pltpu.async_copy(
            u_sp.at[idx_v.at[g % 3, 0]], bufs[g % 2].at[pl.ds(0, GW2)], sem_g)

    def scatter(g):
        return pltpu.async_copy(
            bufs[g % 2].at[pl.ds(0, GW2)], acc.at[idx_v.at[g % 3, 1]],
            ssems[g % 2], add=True)

    def prefetch(g):
        return pltpu.async_copy(
            idx_hbm.at[base + g], idx_v.at[g % 3], isems[g % 3])

    def update_chunk(o, l):
        rr = r0 + o
        pltpu.sync_copy(acc.at[pl.ds(rr, l)], buf0.at[pl.ds(0, l)])
        pltpu.sync_copy(u_sp.at[pl.ds(rr, l)], buf0.at[pl.ds(U1, l)])
        pltpu.sync_copy(u0_sp.at[pl.ds(rr, l)], buf1.at[pl.ds(0, l)])
        pltpu.sync_copy(d2_sp.at[pl.ds(rr, l)], buf1.at[pl.ds(U1, l)])

        def row(i, carry):
            for c in (0, 16):
                a = buf0[i, pl.ds(c, 16)]
                up = buf0[U1 + i, pl.ds(c, 16)]
                u0v = buf1[i, pl.ds(c, 16)]
                d2v = buf1[U1 + i, pl.ds(c, 16)]
                buf0[i, pl.ds(c, 16)] = (
                    (1.0 - ALPHA) * d2v * (a + up) + ALPHA * u0v)
            return carry

        lax.fori_loop(0, l, row, 0)
        pltpu.sync_copy(buf0.at[pl.ds(0, l)], u_sp.at[pl.ds(rr, l)])

    def body(k, carry):
        pltpu.sync_copy(idx_hbm.at[base + 0], idx_v.at[0])
        pltpu.sync_copy(idx_hbm.at[base + 1], idx_v.at[1])
        gd = gather(0)
        sd = [None] * GROUPS2
        pd = [None] * GROUPS2
        for g in range(GROUPS2):
            gd.wait()
            sd[g] = scatter(g)
            if g >= 1:
                sd[g - 1].wait()
            if g + 2 < GROUPS2:
                pd[g + 2] = prefetch(g + 2)
            if g + 1 < GROUPS2:
                if pd[g + 1] is not None:
                    pd[g + 1].wait()
                gd = gather(g + 1)
        sd[GROUPS2 - 1].wait()
        plsc.subcore_barrier()
        update_chunk(0, U1)
        update_chunk(U1, U2)
        plsc.subcore_barrier()
        for b in range(NROWS_TILE // ZB):
            pltpu.sync_copy(z_v, acc.at[pl.ds(sid * NROWS_TILE + b * ZB, ZB)])
        plsc.subcore_barrier()
        return carry

    lax.fori_loop(0, K, body, 0)
    for (o, l) in ((0, U1), (U1, U2)):
        pltpu.sync_copy(u_sp.at[pl.ds(r0 + o, l)], buf0.at[pl.ds(0, l)])
        pltpu.sync_copy(buf0.at[pl.ds(0, l)], out_hbm.at[cid, pl.ds(r0 + o, l)])


def _tc_mm1(x, W1):
    def body(x_ref, w_ref, o_ref):
        o_ref[...] = jnp.dot(x_ref[...], w_ref[...],
                             preferred_element_type=jnp.float32)
    return pl.pallas_call(
        body, out_shape=jax.ShapeDtypeStruct((N, H), jnp.float32))(x, W1)


def _tc_prep(dega, degb, hp):
    def body(da, db, hp_ref, v0, dinv, d2w, sqd):
        deg = da[...] + db[...] + 1.0
        di = lax.rsqrt(deg)
        dinv[...] = di
        d2w[...] = jnp.broadcast_to((1.0 / deg)[:, None], (N, HH))
        sqd[...] = jnp.sqrt(deg)
        v0[...] = hp_ref[...] * di[:, None]
    return pl.pallas_call(
        body,
        out_shape=(
            jax.ShapeDtypeStruct((N, H), jnp.float32),
            jax.ShapeDtypeStruct((N,), jnp.float32),
            jax.ShapeDtypeStruct((N, HH), jnp.float32),
            jax.ShapeDtypeStruct((N,), jnp.float32),
        ))(dega, degb, hp)


def _tc_bn(acc_a, acc_b, v0, dinv, b1, gamma, beta):
    def body(aa, ab, v0r, di, b1r, gr, br, u0r):
        h1 = di[...][:, None] * (aa[...] + ab[...] + v0r[...]) + b1r[...][None, :]
        mean = jnp.mean(h1, axis=0, keepdims=True)
        var = jnp.mean((h1 - mean) ** 2, axis=0, keepdims=True)
        h0 = (h1 - mean) * lax.rsqrt(var + EPS) * gr[...][None, :] + br[...][None, :]
        h0 = jnp.maximum(h0, 0.0)
        u0 = di[...][:, None] * h0
        u0r[0] = u0[:, :HH]
        u0r[1] = u0[:, HH:]
    return pl.pallas_call(
        body, out_shape=jax.ShapeDtypeStruct((NC, N, HH), jnp.float32))(
            acc_a, acc_b, v0, dinv, b1, gamma, beta)


def _tc_combine(acc_a, acc_b, u, u0, dinv2):
    def body(aa, ab, ur, u0r, d2, out):
        out[...] = ((1.0 - ALPHA) * d2[...][:, None]
                    * (aa[...] + ab[...] + ur[...]) + ALPHA * u0r[...])
    return pl.pallas_call(
        body, out_shape=jax.ShapeDtypeStruct((N, H), jnp.float32))(
            acc_a, acc_b, u, u0, dinv2)


def _tc_final(u, sqd, W2, b2):
    def body(ur, sr, w_ref, b_ref, out):
        z = sr[...][:, None] * jnp.concatenate([ur[0], ur[1]], axis=1)
        o = jnp.dot(z, w_ref[...], preferred_element_type=jnp.float32)
        o = o + b_ref[...][None, :]
        o = o - jnp.max(o, axis=1, keepdims=True)
        out[...] = o - jnp.log(jnp.sum(jnp.exp(o), axis=1, keepdims=True))
    return pl.pallas_call(
        body, out_shape=jax.ShapeDtypeStruct((N, C), jnp.float32))(
            u, sqd, W2, b2)


def kernel(x, edge_index, W1, b1, gamma, beta, W2, b2):
    src = edge_index[0]
    dst = edge_index[1]
    pad = EPAD - E
    pad2 = NS * GROUPS2 * GW2 - E
    src2d = jnp.concatenate(
        [src, jnp.zeros((pad,), jnp.int32)]).reshape(-1, GW)
    dst2d = jnp.concatenate(
        [dst, jnp.full((pad,), N, jnp.int32)]).reshape(-1, GW)
    idx3 = jnp.stack([src2d, dst2d], axis=1)
    zrows = jnp.zeros((ZB, H), jnp.float32)
    zeros_t = jnp.zeros((NROWS_TILE,), jnp.float32)
    ones_c = jnp.ones((GW,), jnp.float32)

    src2w = jnp.concatenate(
        [src, jnp.zeros((pad2,), jnp.int32)]).reshape(-1, GW2)
    dst2w = jnp.concatenate(
        [dst, jnp.full((pad2,), N, jnp.int32)]).reshape(-1, GW2)
    idx3w = jnp.stack([src2w, dst2w], axis=1)
    zrows2 = jnp.zeros((ZB, HH), jnp.float32)

    hp = _tc_mm1(x, W1)
    degs = _sc_degree(idx3, ones_c, zeros_t)
    v0, dinv, d2w, sqd = _tc_prep(degs[0, :N], degs[1, :N], hp)
    acc = _sc_propagate(v0, idx3, zrows)
    u0 = _tc_bn(acc[0, :N], acc[1, :N], v0, dinv, b1, gamma, beta)
    uK = _sc_appnp(u0, d2w, idx3w, zrows2)
    return _tc_final(uK, sqd, W2, b2)
